# baseline (speedup 1.0000x reference)
import functools

import jax
import jax.numpy as jnp
from jax import lax
from jax.experimental import pallas as pl
from jax.experimental.pallas import tpu as pltpu
from jax.experimental.pallas import tpu_sc as plsc

_BATCH = 1024
_OUT = 20
_K = 130107
_BK = 2048

_NC = 2
_NS = 16
_NW = _NC * _NS
_KSC = 8192
_KPT = _KSC // _NW
_KCH = 32
_NKC = _KPT // _KCH
_NMC = _BATCH // 16

_CONTRACT = (((1,), (0,)), ((), ()))


def _sc_body(xt_hbm, w_hbm, out_hbm, x_v, w_v, acc_v):
    wid = lax.axis_index("s") * _NC + lax.axis_index("c")
    base = wid * _KPT
    pltpu.sync_copy(w_hbm.at[:, pl.ds(base, _KPT)], w_v)

    def kc_body(kc, _):
        pltpu.sync_copy(xt_hbm.at[pl.ds(base + kc * _KCH, _KCH)], x_v)
        first = kc == 0

        def mc_body(mc, _):
            off = mc * 16
            accs = [
                jnp.where(first, 0.0, acc_v[j, pl.ds(off, 16)])
                for j in range(_OUT)
            ]
            for kg in range(_KCH // 16):
                wv = [
                    w_v[j, pl.ds(kc * _KCH + kg * 16, 16)]
                    for j in range(_OUT)
                ]
                for k in range(16):
                    xv = x_v[kg * 16 + k, pl.ds(off, 16)]
                    for j in range(_OUT):
                        accs[j] = accs[j] + xv * wv[j][k]
            for j in range(_OUT):
                acc_v[j, pl.ds(off, 16)] = accs[j]
            return 0

        lax.fori_loop(0, _NMC, mc_body, 0)
        return 0

    lax.fori_loop(0, _NKC, kc_body, 0)
    pltpu.sync_copy(acc_v, out_hbm.at[wid])


def _sc_partial(Xt, W):
    mesh = plsc.VectorSubcoreMesh(core_axis_name="c", subcore_axis_name="s")
    fn = functools.partial(
        pl.kernel,
        mesh=mesh,
        out_type=jax.ShapeDtypeStruct((_NW, _OUT, _BATCH), jnp.float32),
        scratch_types=[
            pltpu.VMEM((_KCH, _BATCH), jnp.float32),
            pltpu.VMEM((_OUT, _KPT), jnp.float32),
            pltpu.VMEM((_OUT, _BATCH), jnp.float32),
        ],
    )(_sc_body)
    return fn(Xt, W)


def _tc_body(xt_ref, w_ref, o_ref, *, nk, bk, k_total):
    k = pl.program_id(0)

    @pl.when(k == 0)
    def _init():
        o_ref[...] = jnp.zeros_like(o_ref)

    @pl.when(k < nk - 1)
    def _full():
        o_ref[...] += jax.lax.dot_general(
            w_ref[...], xt_ref[...], _CONTRACT,
            preferred_element_type=jnp.float32)

    @pl.when(k == nk - 1)
    def _tail():
        valid = k_total - _KSC - (nk - 1) * bk
        xm = jax.lax.broadcasted_iota(jnp.int32, xt_ref.shape, 0) < valid
        wm = jax.lax.broadcasted_iota(jnp.int32, w_ref.shape, 1) < valid
        x = jnp.where(xm, xt_ref[...], 0.0)
        w = jnp.where(wm, w_ref[...], 0.0)
        o_ref[...] += jax.lax.dot_general(
            w, x, _CONTRACT, preferred_element_type=jnp.float32)


def _tc_partial(Xt, W):
    nk = pl.cdiv(_K - _KSC, _BK)
    off = _KSC // _BK
    body = functools.partial(_tc_body, nk=nk, bk=_BK, k_total=_K)
    return pl.pallas_call(
        body,
        grid=(nk,),
        in_specs=[
            pl.BlockSpec((_BK, _BATCH), lambda k: (k + off, 0)),
            pl.BlockSpec((_OUT, _BK), lambda k: (0, k + off)),
        ],
        out_specs=pl.BlockSpec((_OUT, _BATCH), lambda k: (0, 0)),
        out_shape=jax.ShapeDtypeStruct((_OUT, _BATCH), jnp.float32),
        compiler_params=pltpu.CompilerParams(
            dimension_semantics=("arbitrary",)),
    )(Xt, W)


def _ep_body(tc_ref, sc_ref, b_ref, o_ref):
    logits = tc_ref[...] + jnp.sum(sc_ref[...], axis=0) + b_ref[...]
    m = jnp.max(logits, axis=0, keepdims=True)
    e = jnp.exp(logits - m)
    o_ref[...] = e / jnp.sum(e, axis=0, keepdims=True)


def _epilogue(tc_p, sc_p, b):
    return pl.pallas_call(
        _ep_body,
        out_shape=jax.ShapeDtypeStruct((_OUT, _BATCH), jnp.float32),
    )(tc_p, sc_p, b.reshape(_OUT, 1))


def kernel(X, W, b):
    Xt = X.T
    sc_p = _sc_partial(Xt, W)
    tc_p = _tc_partial(Xt, W)
    return _epilogue(tc_p, sc_p, b).T


# speedup vs baseline: 1.4277x; 1.4277x over previous
import functools

import jax
import jax.numpy as jnp
from jax import lax
from jax.experimental import pallas as pl
from jax.experimental.pallas import tpu as pltpu
from jax.experimental.pallas import tpu_sc as plsc

_BATCH = 1024
_OUT = 20
_K = 130107
_BK = 2048

_NC = 2
_NS = 16
_NW = _NC * _NS
_KSC = 4096
_KPT = _KSC // _NW
_KCH = 32
_NKC = _KPT // _KCH
_NMC = _BATCH // 16

_CONTRACT = (((1,), (0,)), ((), ()))


def _sc_body(xt_hbm, w_hbm, out_hbm, x_v, w_v, acc_v):
    wid = lax.axis_index("s") * _NC + lax.axis_index("c")
    base = wid * _KPT
    pltpu.sync_copy(w_hbm.at[:, pl.ds(base, _KPT)], w_v)

    def kc_body(kc, _):
        pltpu.sync_copy(xt_hbm.at[pl.ds(base + kc * _KCH, _KCH)], x_v)
        first = kc == 0

        def mc_body(mc, _):
            off = mc * 16
            accs = [
                jnp.where(first, 0.0, acc_v[j, pl.ds(off, 16)])
                for j in range(_OUT)
            ]
            for kg in range(_KCH // 16):
                wv = [
                    w_v[j, pl.ds(kc * _KCH + kg * 16, 16)]
                    for j in range(_OUT)
                ]
                for k in range(16):
                    xv = x_v[kg * 16 + k, pl.ds(off, 16)]
                    for j in range(_OUT):
                        accs[j] = accs[j] + xv * wv[j][k]
            for j in range(_OUT):
                acc_v[j, pl.ds(off, 16)] = accs[j]
            return 0

        lax.fori_loop(0, _NMC, mc_body, 0)
        return 0

    lax.fori_loop(0, _NKC, kc_body, 0)
    pltpu.sync_copy(acc_v, out_hbm.at[wid])


def _sc_partial(Xt, W):
    mesh = plsc.VectorSubcoreMesh(core_axis_name="c", subcore_axis_name="s")
    fn = functools.partial(
        pl.kernel,
        mesh=mesh,
        out_type=jax.ShapeDtypeStruct((_NW, _OUT, _BATCH), jnp.float32),
        scratch_types=[
            pltpu.VMEM((_KCH, _BATCH), jnp.float32),
            pltpu.VMEM((_OUT, _KPT), jnp.float32),
            pltpu.VMEM((_OUT, _BATCH), jnp.float32),
        ],
    )(_sc_body)
    return fn(Xt, W)


def _tc_body(xt_ref, w_ref, o_ref, *, nk, bk, k_total):
    k = pl.program_id(0)

    @pl.when(k == 0)
    def _init():
        o_ref[...] = jnp.zeros_like(o_ref)

    @pl.when(k < nk - 1)
    def _full():
        o_ref[...] += jax.lax.dot_general(
            w_ref[...], xt_ref[...], _CONTRACT,
            preferred_element_type=jnp.float32)

    @pl.when(k == nk - 1)
    def _tail():
        valid = k_total - _KSC - (nk - 1) * bk
        xm = jax.lax.broadcasted_iota(jnp.int32, xt_ref.shape, 0) < valid
        wm = jax.lax.broadcasted_iota(jnp.int32, w_ref.shape, 1) < valid
        x = jnp.where(xm, xt_ref[...], 0.0)
        w = jnp.where(wm, w_ref[...], 0.0)
        o_ref[...] += jax.lax.dot_general(
            w, x, _CONTRACT, preferred_element_type=jnp.float32)


def _tc_partial(Xt, W):
    nk = pl.cdiv(_K - _KSC, _BK)
    off = _KSC // _BK
    body = functools.partial(_tc_body, nk=nk, bk=_BK, k_total=_K)
    return pl.pallas_call(
        body,
        grid=(nk,),
        in_specs=[
            pl.BlockSpec((_BK, _BATCH), lambda k: (k + off, 0)),
            pl.BlockSpec((_OUT, _BK), lambda k: (0, k + off)),
        ],
        out_specs=pl.BlockSpec((_OUT, _BATCH), lambda k: (0, 0)),
        out_shape=jax.ShapeDtypeStruct((_OUT, _BATCH), jnp.float32),
        compiler_params=pltpu.CompilerParams(
            dimension_semantics=("arbitrary",)),
    )(Xt, W)


def _ep_body(tc_ref, sc_ref, b_ref, o_ref):
    logits = tc_ref[...] + jnp.sum(sc_ref[...], axis=0) + b_ref[...]
    m = jnp.max(logits, axis=0, keepdims=True)
    e = jnp.exp(logits - m)
    o_ref[...] = e / jnp.sum(e, axis=0, keepdims=True)


def _epilogue(tc_p, sc_p, b):
    return pl.pallas_call(
        _ep_body,
        out_shape=jax.ShapeDtypeStruct((_OUT, _BATCH), jnp.float32),
    )(tc_p, sc_p, b.reshape(_OUT, 1))


def kernel(X, W, b):
    Xt = X.T
    sc_p = _sc_partial(Xt, W)
    tc_p = _tc_partial(Xt, W)
    return _epilogue(tc_p, sc_p, b).T


# speedup vs baseline: 1.5912x; 1.1145x over previous
import functools

import jax
import jax.numpy as jnp
from jax.experimental import pallas as pl
from jax.experimental.pallas import tpu as pltpu

_BATCH = 1024
_OUT = 20
_K = 130107
_BK = 2560

_CONTRACT = (((1,), (0,)), ((), ()))


def _body(xt_ref, w_ref, b_ref, o_ref, *, nk, bk, k_total):
    k = pl.program_id(0)

    @pl.when(k == 0)
    def _init():
        o_ref[...] = jnp.zeros_like(o_ref)

    @pl.when(k < nk - 1)
    def _full():
        o_ref[...] += jax.lax.dot_general(
            w_ref[...], xt_ref[...], _CONTRACT,
            preferred_element_type=jnp.float32)

    @pl.when(k == nk - 1)
    def _tail():
        valid = k_total - (nk - 1) * bk
        xm = jax.lax.broadcasted_iota(jnp.int32, xt_ref.shape, 0) < valid
        wm = jax.lax.broadcasted_iota(jnp.int32, w_ref.shape, 1) < valid
        x = jnp.where(xm, xt_ref[...], 0.0)
        w = jnp.where(wm, w_ref[...], 0.0)
        logits = o_ref[...] + jax.lax.dot_general(
            w, x, _CONTRACT, preferred_element_type=jnp.float32)
        logits += b_ref[...]
        m = jnp.max(logits, axis=0, keepdims=True)
        e = jnp.exp(logits - m)
        o_ref[...] = e / jnp.sum(e, axis=0, keepdims=True)


def kernel(X, W, b):
    nk = pl.cdiv(_K, _BK)
    body = functools.partial(_body, nk=nk, bk=_BK, k_total=_K)
    out_t = pl.pallas_call(
        body,
        grid=(nk,),
        in_specs=[
            pl.BlockSpec((_BK, _BATCH), lambda k: (k, 0)),
            pl.BlockSpec((_OUT, _BK), lambda k: (0, k)),
            pl.BlockSpec((_OUT, 1), lambda k: (0, 0)),
        ],
        out_specs=pl.BlockSpec((_OUT, _BATCH), lambda k: (0, 0)),
        out_shape=jax.ShapeDtypeStruct((_OUT, _BATCH), jnp.float32),
        compiler_params=pltpu.CompilerParams(
            dimension_semantics=("arbitrary",)),
    )(X.T, W, b.reshape(_OUT, 1))
    return out_t.T
